# baseline (speedup 1.0000x reference)
import functools

import jax
import jax.numpy as jnp
from jax import lax
from jax.experimental import pallas as pl
from jax.experimental.pallas import tpu as pltpu
from jax.experimental.pallas import tpu_sc as plsc

_B = 4096
_T = 200
_D = 64
_L = 16
_NC = 2
_NS = 16
_NW = _NC * _NS
_BPW = _B // _NW
_NCHUNK = (_T + _L - 1) // _L


def _pool_body(x_hbm, len_hbm, tab_hbm, out_hbm,
               xv, lenv, xpad, rowbuf, row0v, outv, sem):
    wid = lax.axis_index("s") * _NC + lax.axis_index("c")
    base = wid * _BPW
    pltpu.sync_copy(x_hbm.at[pl.ds(base, _BPW)], xv)
    pltpu.sync_copy(len_hbm.at[pl.ds(base, _BPW)], lenv)
    pltpu.sync_copy(tab_hbm.at[pl.ds(0, 1)], row0v)

    iota16 = lax.broadcasted_iota(jnp.int32, (_L,), 0)

    def row_body(b, carry):
        g = b // _L
        j = b - g * _L
        lvec = lenv[pl.ds(g * _L, _L)]
        lb = jnp.max(jnp.where(iota16 == j, lvec, 0))

        for c in range(_NCHUNK):
            start = c * _L if c < _NCHUNK - 1 else _T - _L
            tv = iota16 + start
            valid = tv < lb
            if c == _NCHUNK - 1:
                valid = jnp.logical_and(valid, tv >= (_NCHUNK - 1) * _L)
            xc = xv[b, pl.ds(start, _L)]
            xpad[pl.ds(c * _L, _L)] = jnp.where(valid, xc, 0)

        nch = (lb + _L - 1) // _L

        def chunk(cb, acc):
            cp = pltpu.async_copy(
                tab_hbm.at[xpad.at[pl.ds(cb * _L, _L)]], rowbuf, sem)
            cp.wait()
            a0, a1, a2, a3 = acc
            for r in range(_L):
                a0 = a0 + rowbuf[r, pl.ds(0, _L)]
                a1 = a1 + rowbuf[r, pl.ds(_L, _L)]
                a2 = a2 + rowbuf[r, pl.ds(2 * _L, _L)]
                a3 = a3 + rowbuf[r, pl.ds(3 * _L, _L)]
            return (a0, a1, a2, a3)

        zero = jnp.zeros((_L,), jnp.float32)
        acc = lax.fori_loop(0, nch, chunk, (zero, zero, zero, zero))

        padv = jnp.full((_L,), nch * _L - lb, jnp.int32).astype(jnp.float32)
        denv = jnp.maximum(jnp.full((_L,), lb, jnp.int32), 1).astype(jnp.float32)
        rec = 1.0 / denv
        for d in range(_D // _L):
            outv[b, pl.ds(d * _L, _L)] = (
                acc[d] - padv * row0v[0, pl.ds(d * _L, _L)]) * rec
        return carry

    lax.fori_loop(0, _BPW, row_body, 0)
    pltpu.sync_copy(outv, out_hbm.at[pl.ds(base, _BPW)])


_pool = functools.partial(
    pl.kernel,
    out_type=jax.ShapeDtypeStruct((_B, _D), jnp.float32),
    mesh=plsc.VectorSubcoreMesh(core_axis_name="c", subcore_axis_name="s"),
    compiler_params=pltpu.CompilerParams(
        needs_layout_passes=False, use_tc_tiling_on_sc=False),
    scratch_types=[
        pltpu.VMEM((_BPW, _T), jnp.int32),
        pltpu.VMEM((_BPW,), jnp.int32),
        pltpu.VMEM((_NCHUNK * _L,), jnp.int32),
        pltpu.VMEM((_L, _D), jnp.float32),
        pltpu.VMEM((1, _D), jnp.float32),
        pltpu.VMEM((_BPW, _D), jnp.float32),
        pltpu.SemaphoreType.DMA,
    ],
)(_pool_body)


def _head_body(ncls, p_ref, w_ref, b_ref, o_ref):
    logits = jnp.dot(p_ref[...], w_ref[...],
                     preferred_element_type=jnp.float32) + b_ref[...]
    col = lax.broadcasted_iota(jnp.int32, logits.shape, 1)
    masked = jnp.where(col < ncls, logits, -1e30)
    m = jnp.max(masked, axis=1, keepdims=True)
    sh = masked - m
    s = jnp.sum(jnp.exp(sh), axis=1, keepdims=True)
    o_ref[...] = sh - jnp.log(s)


def kernel(x, lengths, table, W_out, b_out):
    x = x.astype(jnp.int32)
    lengths = lengths.astype(jnp.int32)
    pooled = _pool(x, lengths, table)

    ncls = W_out.shape[1]
    wp = jnp.zeros((_D, 128), jnp.float32).at[:, :ncls].set(W_out)
    bp = jnp.zeros((1, 128), jnp.float32).at[0, :ncls].set(b_out)
    out = pl.pallas_call(
        functools.partial(_head_body, ncls),
        out_shape=jax.ShapeDtypeStruct((_B, 128), jnp.float32),
    )(pooled, wp, bp)
    return out[:, :ncls]


# speedup vs baseline: 1.0422x; 1.0422x over previous
import functools

import jax
import jax.numpy as jnp
from jax import lax
from jax.experimental import pallas as pl
from jax.experimental.pallas import tpu as pltpu
from jax.experimental.pallas import tpu_sc as plsc

_B = 4096
_T = 200
_D = 64
_L = 16
_NC = 2
_NS = 16
_NW = _NC * _NS
_BPW = _B // _NW
_NCHUNK = (_T + _L - 1) // _L
_ND = _D // _L


def _pool_body(x_hbm, len_hbm, tab_hbm, out_hbm,
               xv, lenv, xpad, dst, row0v, outv, sem0, sem1):
    wid = lax.axis_index("s") * _NC + lax.axis_index("c")
    base = wid * _BPW
    pltpu.sync_copy(x_hbm.at[pl.ds(base, _BPW)], xv)
    pltpu.sync_copy(len_hbm.at[pl.ds(base, _BPW)], lenv)
    pltpu.sync_copy(tab_hbm.at[pl.ds(0, 1)], row0v)

    iota16 = lax.broadcasted_iota(jnp.int32, (_L,), 0)
    zerov = jnp.zeros((_L,), jnp.float32)
    sems = (sem0, sem1)

    for s in range(2):
        for r in range(_L):
            for d in range(_ND):
                dst[s, r, pl.ds(d * _L, _L)] = zerov

    def issue(b, slot):
        g = b // _L
        j = b - g * _L
        lvec = lenv[pl.ds(g * _L, _L)]
        lb = jnp.max(jnp.where(iota16 == j, lvec, 0))
        for c in range(_NCHUNK):
            start = c * _L if c < _NCHUNK - 1 else _T - _L
            tv = iota16 + start
            valid = tv < lb
            if c == _NCHUNK - 1:
                valid = jnp.logical_and(valid, tv >= (_NCHUNK - 1) * _L)
            xc = xv[b, pl.ds(start, _L)]
            xpad[slot, pl.ds(c * _L, _L)] = jnp.where(valid, xc, 0)
        nch = (lb + _L - 1) // _L

        def chunk(cb, carry):
            pltpu.async_copy(
                tab_hbm.at[xpad.at[slot, pl.ds(cb * _L, _L)]],
                dst.at[slot], sems[slot], add=True)
            return carry

        lax.fori_loop(0, nch, chunk, 0)
        return lb

    def finish(b, slot, lb):
        nch = (lb + _L - 1) // _L

        def drain(cb, carry):
            pltpu.make_async_copy(
                tab_hbm.at[pl.ds(0, _L)], dst.at[slot], sems[slot]).wait()
            return carry

        lax.fori_loop(0, nch, drain, 0)
        acc = [zerov] * _ND
        for r in range(_L):
            for d in range(_ND):
                acc[d] = acc[d] + dst[slot, r, pl.ds(d * _L, _L)]
                dst[slot, r, pl.ds(d * _L, _L)] = zerov
        padv = jnp.full((_L,), nch * _L - lb, jnp.int32).astype(jnp.float32)
        denv = jnp.maximum(jnp.full((_L,), lb, jnp.int32), 1).astype(
            jnp.float32)
        rec = 1.0 / denv
        for d in range(_ND):
            outv[b, pl.ds(d * _L, _L)] = (
                acc[d] - padv * row0v[0, pl.ds(d * _L, _L)]) * rec

    lb0 = issue(0, 0)

    def pipe2(i, lb_even):
        b = 2 * i
        lb_odd = issue(b + 1, 1)
        finish(b, 0, lb_even)
        lb_next = issue(b + 2, 0)
        finish(b + 1, 1, lb_odd)
        return lb_next

    lb_last = lax.fori_loop(0, (_BPW - 2) // 2, pipe2, lb0)
    lb_127 = issue(_BPW - 1, 1)
    finish(_BPW - 2, 0, lb_last)
    finish(_BPW - 1, 1, lb_127)

    pltpu.sync_copy(outv, out_hbm.at[pl.ds(base, _BPW)])


_pool = functools.partial(
    pl.kernel,
    out_type=jax.ShapeDtypeStruct((_B, _D), jnp.float32),
    mesh=plsc.VectorSubcoreMesh(core_axis_name="c", subcore_axis_name="s"),
    compiler_params=pltpu.CompilerParams(
        needs_layout_passes=False, use_tc_tiling_on_sc=False),
    scratch_types=[
        pltpu.VMEM((_BPW, _T), jnp.int32),
        pltpu.VMEM((_BPW,), jnp.int32),
        pltpu.VMEM((2, _NCHUNK * _L), jnp.int32),
        pltpu.VMEM((2, _L, _D), jnp.float32),
        pltpu.VMEM((1, _D), jnp.float32),
        pltpu.VMEM((_BPW, _D), jnp.float32),
        pltpu.SemaphoreType.DMA,
        pltpu.SemaphoreType.DMA,
    ],
)(_pool_body)


def _head_body(ncls, p_ref, w_ref, b_ref, o_ref):
    logits = jnp.dot(p_ref[...], w_ref[...],
                     preferred_element_type=jnp.float32) + b_ref[...]
    col = lax.broadcasted_iota(jnp.int32, logits.shape, 1)
    masked = jnp.where(col < ncls, logits, -1e30)
    m = jnp.max(masked, axis=1, keepdims=True)
    sh = masked - m
    s = jnp.sum(jnp.exp(sh), axis=1, keepdims=True)
    o_ref[...] = sh - jnp.log(s)


def kernel(x, lengths, table, W_out, b_out):
    x = x.astype(jnp.int32)
    lengths = lengths.astype(jnp.int32)
    pooled = _pool(x, lengths, table)

    ncls = W_out.shape[1]
    wp = jnp.zeros((_D, 128), jnp.float32).at[:, :ncls].set(W_out)
    bp = jnp.zeros((1, 128), jnp.float32).at[0, :ncls].set(b_out)
    out = pl.pallas_call(
        functools.partial(_head_body, ncls),
        out_shape=jax.ShapeDtypeStruct((_B, 128), jnp.float32),
    )(pooled, wp, bp)
    return out[:, :ncls]


# speedup vs baseline: 3.1947x; 3.0653x over previous
import functools

import jax
import jax.numpy as jnp
from jax import lax
from jax.experimental import pallas as pl
from jax.experimental.pallas import tpu as pltpu
from jax.experimental.pallas import tpu_sc as plsc

_B = 4096
_T = 200
_D = 64
_V = 1000000
_L = 16
_NC = 2
_NS = 16
_NW = _NC * _NS
_BPW = _B // _NW
_NCHUNK = (_T + _L - 1) // _L
_BLK = 4096
_GRID = (_V + _BLK - 1) // _BLK
_NS_PACK = _BLK // 128
_TWROWS = _GRID * _BLK


def _proj_body(tt_ref, w_ref, o_ref):
    res = lax.dot_general(
        tt_ref[...], w_ref[...],
        dimension_numbers=(((0,), (0,)), ((), ())),
        preferred_element_type=jnp.float32)
    for s in range(8):
        o_ref[:, s * _L:(s + 1) * _L] = res[512 * s:512 * (s + 1), :]


_proj = pl.pallas_call(
    _proj_body,
    grid=(_GRID,),
    in_specs=[
        pl.BlockSpec((_D, _BLK), lambda i: (0, i)),
        pl.BlockSpec((_D, _L), lambda i: (0, 0)),
    ],
    out_specs=pl.BlockSpec((512, 128), lambda i: (i, 0)),
    out_shape=jax.ShapeDtypeStruct((_GRID * 512, 128), jnp.float32),
)


def _pool_body(x_hbm, len_hbm, tw_hbm, out_hbm, xv, lenv, dst, outv,
               sem0, sem1):
    wid = lax.axis_index("s") * _NC + lax.axis_index("c")
    base = wid * _BPW
    pltpu.sync_copy(x_hbm.at[pl.ds(base, _BPW)], xv)
    pltpu.sync_copy(len_hbm.at[pl.ds(base, _BPW)], lenv)

    iota16 = lax.broadcasted_iota(jnp.int32, (_L,), 0)
    zerov = jnp.zeros((_L,), jnp.float32)
    sems = (sem0, sem1)

    def issue(b, slot):
        g = b // _L
        j = b - g * _L
        lvec = lenv[pl.ds(g * _L, _L)]
        lb = jnp.max(jnp.where(iota16 == j, lvec, 0))
        nch = (lb + _L - 1) // _L

        def chunk(cb, carry):
            start = jnp.minimum(cb * _L, _T - _L)
            xc = xv[b, pl.ds(start, _L)]
            row16 = (
                ((xc >> 12) << 12)
                | ((xc & 511) << 3)
                | ((xc & 4095) >> 9))
            pltpu.async_copy(tw_hbm.at[row16], dst.at[slot, cb], sems[slot])
            return carry

        lax.fori_loop(0, nch, chunk, 0)
        return lb

    def finish(b, slot, lb):
        nch = (lb + _L - 1) // _L

        def drain(cb, carry):
            pltpu.make_async_copy(
                tw_hbm.at[pl.ds(0, _L)], dst.at[slot, 0], sems[slot]).wait()
            return carry

        lax.fori_loop(0, nch, drain, 0)

        nfull = jnp.minimum(lb // _L, _NCHUNK - 1)
        rem_off = jnp.where(
            lb >= (_NCHUNK - 1) * _L, (_NCHUNK - 1) * _L - (_T - _L), 0)
        rem_cnt = lb - nfull * _L

        def accum(cb, acc):
            for r in range(_L):
                acc = acc + dst[slot, cb, r, pl.ds(0, _L)]
            return acc

        acc = lax.fori_loop(0, nfull, accum, zerov)

        def accum_tail(r, acc):
            return acc + dst[slot, nfull, rem_off + r, pl.ds(0, _L)]

        acc = lax.fori_loop(0, rem_cnt, accum_tail, acc)

        denv = jnp.maximum(jnp.full((_L,), lb, jnp.int32), 1).astype(
            jnp.float32)
        outv[b, pl.ds(0, _L)] = acc / denv

    lb0 = issue(0, 0)

    def pipe2(i, lb_even):
        b = 2 * i
        lb_odd = issue(b + 1, 1)
        finish(b, 0, lb_even)
        lb_next = issue(b + 2, 0)
        finish(b + 1, 1, lb_odd)
        return lb_next

    lb_last = lax.fori_loop(0, (_BPW - 2) // 2, pipe2, lb0)
    lb_127 = issue(_BPW - 1, 1)
    finish(_BPW - 2, 0, lb_last)
    finish(_BPW - 1, 1, lb_127)

    pltpu.sync_copy(outv, out_hbm.at[pl.ds(base, _BPW)])


_pool = functools.partial(
    pl.kernel,
    out_type=jax.ShapeDtypeStruct((_B, _L), jnp.float32),
    mesh=plsc.VectorSubcoreMesh(core_axis_name="c", subcore_axis_name="s"),
    compiler_params=pltpu.CompilerParams(
        needs_layout_passes=False, use_tc_tiling_on_sc=False),
    scratch_types=[
        pltpu.VMEM((_BPW, _T), jnp.int32),
        pltpu.VMEM((_BPW,), jnp.int32),
        pltpu.VMEM((2, _NCHUNK, _L, _L), jnp.float32),
        pltpu.VMEM((_BPW, _L), jnp.float32),
        pltpu.SemaphoreType.DMA,
        pltpu.SemaphoreType.DMA,
    ],
)(_pool_body)


def _head_body(ncls, p_ref, b_ref, o_ref):
    logits = p_ref[...] + b_ref[...]
    col = lax.broadcasted_iota(jnp.int32, logits.shape, 1)
    masked = jnp.where(col < ncls, logits, -1e30)
    m = jnp.max(masked, axis=1, keepdims=True)
    sh = masked - m
    s = jnp.sum(jnp.exp(sh), axis=1, keepdims=True)
    o_ref[...] = sh - jnp.log(s)


def kernel(x, lengths, table, W_out, b_out):
    x = x.astype(jnp.int32)
    lengths = lengths.astype(jnp.int32)
    ncls = W_out.shape[1]

    wp = jnp.zeros((_D, _L), jnp.float32).at[:, :ncls].set(W_out)
    twp = _proj(jnp.swapaxes(table, 0, 1), wp)
    tw = twp.reshape(_TWROWS, _L)

    pooled = _pool(x, lengths, tw)

    bp = jnp.zeros((1, _L), jnp.float32).at[0, :ncls].set(b_out)
    out = pl.pallas_call(
        functools.partial(_head_body, ncls),
        out_shape=jax.ShapeDtypeStruct((_B, _L), jnp.float32),
    )(pooled, bp)
    return out[:, :ncls]


# speedup vs baseline: 3.5844x; 1.1220x over previous
import functools

import jax
import jax.numpy as jnp
from jax import lax
from jax.experimental import pallas as pl
from jax.experimental.pallas import tpu as pltpu
from jax.experimental.pallas import tpu_sc as plsc

_B = 4096
_T = 200
_D = 64
_V = 1000000
_L = 16
_NC = 2
_NS = 16
_NW = _NC * _NS
_BPW = _B // _NW
_NCHUNK = (_T + _L - 1) // _L
_BLK = 4096
_GRID = (_V + _BLK - 1) // _BLK
_NS_PACK = _BLK // 128
_TWROWS = _GRID * _BLK


def _proj_body(tt_ref, w_ref, o_ref):
    acc = jnp.zeros((512, 128), jnp.float32)
    for s in range(8):
        acc = acc + lax.dot_general(
            tt_ref[:, 512 * s:512 * (s + 1)], w_ref[:, 128 * s:128 * (s + 1)],
            dimension_numbers=(((0,), (0,)), ((), ())),
            preferred_element_type=jnp.float32)
    o_ref[...] = acc


_proj = pl.pallas_call(
    _proj_body,
    grid=(_GRID,),
    in_specs=[
        pl.BlockSpec((_D, _BLK), lambda i: (0, i)),
        pl.BlockSpec((_D, 1024), lambda i: (0, 0)),
    ],
    out_specs=pl.BlockSpec((512, 128), lambda i: (i, 0)),
    out_shape=jax.ShapeDtypeStruct((_GRID * 512, 128), jnp.float32),
    compiler_params=pltpu.CompilerParams(fuse_transposed_lhs_in_matmul=True),
)


def _pool_body(x_hbm, len_hbm, tw_hbm, out_hbm, xv, lenv, dst, outv,
               sem0, sem1):
    wid = lax.axis_index("s") * _NC + lax.axis_index("c")
    base = wid * _BPW
    pltpu.sync_copy(x_hbm.at[pl.ds(base, _BPW)], xv)
    pltpu.sync_copy(len_hbm.at[pl.ds(base, _BPW)], lenv)

    iota16 = lax.broadcasted_iota(jnp.int32, (_L,), 0)
    zerov = jnp.zeros((_L,), jnp.float32)
    sems = (sem0, sem1)

    def issue(b, slot):
        g = b // _L
        j = b - g * _L
        lvec = lenv[pl.ds(g * _L, _L)]
        lb = jnp.max(jnp.where(iota16 == j, lvec, 0))
        nch = (lb + _L - 1) // _L

        def chunk(cb, carry):
            start = jnp.minimum(cb * _L, _T - _L)
            xc = xv[b, pl.ds(start, _L)]
            row16 = (
                ((xc >> 12) << 12)
                | ((xc & 511) << 3)
                | ((xc & 4095) >> 9))
            pltpu.async_copy(tw_hbm.at[row16], dst.at[slot, cb], sems[slot])
            return carry

        lax.fori_loop(0, nch, chunk, 0)
        return lb

    def finish(b, slot, lb):
        nch = (lb + _L - 1) // _L

        def drain(cb, carry):
            pltpu.make_async_copy(
                tw_hbm.at[pl.ds(0, _L)], dst.at[slot, 0], sems[slot]).wait()
            return carry

        lax.fori_loop(0, nch, drain, 0)

        nfull = jnp.minimum(lb // _L, _NCHUNK - 1)
        rem_off = jnp.where(
            lb >= (_NCHUNK - 1) * _L, (_NCHUNK - 1) * _L - (_T - _L), 0)
        rem_cnt = lb - nfull * _L

        def accum(cb, acc):
            for r in range(_L):
                acc = acc + dst[slot, cb, r, pl.ds(0, _L)]
            return acc

        acc = lax.fori_loop(0, nfull, accum, zerov)

        def accum_tail(r, acc):
            return acc + dst[slot, nfull, rem_off + r, pl.ds(0, _L)]

        acc = lax.fori_loop(0, rem_cnt, accum_tail, acc)

        denv = jnp.maximum(jnp.full((_L,), lb, jnp.int32), 1).astype(
            jnp.float32)
        outv[b, pl.ds(0, _L)] = acc / denv

    lb0 = issue(0, 0)

    def pipe2(i, lb_even):
        b = 2 * i
        lb_odd = issue(b + 1, 1)
        finish(b, 0, lb_even)
        lb_next = issue(b + 2, 0)
        finish(b + 1, 1, lb_odd)
        return lb_next

    lb_last = lax.fori_loop(0, (_BPW - 2) // 2, pipe2, lb0)
    lb_127 = issue(_BPW - 1, 1)
    finish(_BPW - 2, 0, lb_last)
    finish(_BPW - 1, 1, lb_127)

    pltpu.sync_copy(outv, out_hbm.at[pl.ds(base, _BPW)])


_pool = functools.partial(
    pl.kernel,
    out_type=jax.ShapeDtypeStruct((_B, _L), jnp.float32),
    mesh=plsc.VectorSubcoreMesh(core_axis_name="c", subcore_axis_name="s"),
    compiler_params=pltpu.CompilerParams(
        needs_layout_passes=False, use_tc_tiling_on_sc=False),
    scratch_types=[
        pltpu.VMEM((_BPW, _T), jnp.int32),
        pltpu.VMEM((_BPW,), jnp.int32),
        pltpu.VMEM((2, _NCHUNK, _L, _L), jnp.float32),
        pltpu.VMEM((_BPW, _L), jnp.float32),
        pltpu.SemaphoreType.DMA,
        pltpu.SemaphoreType.DMA,
    ],
)(_pool_body)


def _head_body(ncls, p_ref, b_ref, o_ref):
    logits = p_ref[...] + b_ref[...]
    col = lax.broadcasted_iota(jnp.int32, logits.shape, 1)
    masked = jnp.where(col < ncls, logits, -1e30)
    m = jnp.max(masked, axis=1, keepdims=True)
    sh = masked - m
    s = jnp.sum(jnp.exp(sh), axis=1, keepdims=True)
    o_ref[...] = sh - jnp.log(s)


def kernel(x, lengths, table, W_out, b_out):
    x = x.astype(jnp.int32)
    lengths = lengths.astype(jnp.int32)
    ncls = W_out.shape[1]

    wp = jnp.zeros((_D, 8, 128), jnp.float32)
    for s in range(8):
        wp = wp.at[:, s, _L * s:_L * s + ncls].set(W_out)
    twp = _proj(jnp.swapaxes(table, 0, 1), wp.reshape(_D, 1024))
    tw = twp.reshape(_TWROWS, _L)

    pooled = _pool(x, lengths, tw)

    bp = jnp.zeros((1, _L), jnp.float32).at[0, :ncls].set(b_out)
    out = pl.pallas_call(
        functools.partial(_head_body, ncls),
        out_shape=jax.ShapeDtypeStruct((_B, _L), jnp.float32),
    )(pooled, bp)
    return out[:, :ncls]


# speedup vs baseline: 4.4113x; 1.2307x over previous
import functools

import jax
import jax.numpy as jnp
from jax import lax
from jax.experimental import pallas as pl
from jax.experimental.pallas import tpu as pltpu
from jax.experimental.pallas import tpu_sc as plsc

_B = 4096
_T = 200
_D = 64
_V = 1000000
_L = 16
_NC = 2
_NS = 16
_NW = _NC * _NS
_BPW = _B // _NW
_NCHUNK = (_T + _L - 1) // _L
_BLK = 8192
_GRID = (_V + _BLK - 1) // _BLK
_PIECE = _BLK // 8
_TWROWS = _GRID * _BLK


def _proj_body(tt_ref, w_ref, o_ref):
    pieces = [
        lax.dot_general(
            tt_ref[:, _PIECE * s:_PIECE * (s + 1)],
            w_ref[:, 128 * s:128 * (s + 1)],
            dimension_numbers=(((0,), (0,)), ((), ())),
            preferred_element_type=jnp.float32)
        for s in range(8)
    ]
    while len(pieces) > 1:
        pieces = [pieces[k] + pieces[k + 1] for k in range(0, len(pieces), 2)]
    o_ref[...] = pieces[0]


_proj = pl.pallas_call(
    _proj_body,
    grid=(_GRID,),
    in_specs=[
        pl.BlockSpec((_D, _BLK), lambda i: (0, i)),
        pl.BlockSpec((_D, 1024), lambda i: (0, 0)),
    ],
    out_specs=pl.BlockSpec((_PIECE, 128), lambda i: (i, 0)),
    out_shape=jax.ShapeDtypeStruct((_GRID * _PIECE, 128), jnp.float32),
    compiler_params=pltpu.CompilerParams(fuse_transposed_lhs_in_matmul=True),
)


def _pool_body(x_hbm, len_hbm, tw_hbm, out_hbm, xv, lenv, dst, outv,
               sem0, sem1):
    wid = lax.axis_index("s") * _NC + lax.axis_index("c")
    base = wid * _BPW
    pltpu.sync_copy(x_hbm.at[pl.ds(base, _BPW)], xv)
    pltpu.sync_copy(len_hbm.at[pl.ds(base, _BPW)], lenv)

    iota16 = lax.broadcasted_iota(jnp.int32, (_L,), 0)
    zerov = jnp.zeros((_L,), jnp.float32)
    sems = (sem0, sem1)

    def issue(b, slot):
        g = b // _L
        j = b - g * _L
        lvec = lenv[pl.ds(g * _L, _L)]
        lb = jnp.max(jnp.where(iota16 == j, lvec, 0))
        nch = (lb + _L - 1) // _L

        def chunk(cb, carry):
            start = jnp.minimum(cb * _L, _T - _L)
            xc = xv[b, pl.ds(start, _L)]
            row16 = (
                ((xc >> 13) << 13)
                | ((xc & 1023) << 3)
                | ((xc & 8191) >> 10))
            pltpu.async_copy(tw_hbm.at[row16], dst.at[slot, cb], sems[slot])
            return carry

        lax.fori_loop(0, nch, chunk, 0)
        return lb

    def finish(b, slot, lb):
        nch = (lb + _L - 1) // _L

        def drain(cb, carry):
            pltpu.make_async_copy(
                tw_hbm.at[pl.ds(0, _L)], dst.at[slot, 0], sems[slot]).wait()
            return carry

        lax.fori_loop(0, nch, drain, 0)

        nfull = jnp.minimum(lb // _L, _NCHUNK - 1)
        rem_off = jnp.where(
            lb >= (_NCHUNK - 1) * _L, (_NCHUNK - 1) * _L - (_T - _L), 0)
        rem_cnt = lb - nfull * _L

        def accum(cb, acc):
            for r in range(_L):
                acc = acc + dst[slot, cb, r, pl.ds(0, _L)]
            return acc

        acc = lax.fori_loop(0, nfull, accum, zerov)

        def accum_tail(r, acc):
            return acc + dst[slot, nfull, rem_off + r, pl.ds(0, _L)]

        acc = lax.fori_loop(0, rem_cnt, accum_tail, acc)

        denv = jnp.maximum(jnp.full((_L,), lb, jnp.int32), 1).astype(
            jnp.float32)
        outv[b, pl.ds(0, _L)] = acc / denv

    lb0 = issue(0, 0)

    def pipe2(i, lb_even):
        b = 2 * i
        lb_odd = issue(b + 1, 1)
        finish(b, 0, lb_even)
        lb_next = issue(b + 2, 0)
        finish(b + 1, 1, lb_odd)
        return lb_next

    lb_last = lax.fori_loop(0, (_BPW - 2) // 2, pipe2, lb0)
    lb_127 = issue(_BPW - 1, 1)
    finish(_BPW - 2, 0, lb_last)
    finish(_BPW - 1, 1, lb_127)

    pltpu.sync_copy(outv, out_hbm.at[pl.ds(base, _BPW)])


_pool = functools.partial(
    pl.kernel,
    out_type=jax.ShapeDtypeStruct((_B, _L), jnp.float32),
    mesh=plsc.VectorSubcoreMesh(core_axis_name="c", subcore_axis_name="s"),
    compiler_params=pltpu.CompilerParams(
        needs_layout_passes=False, use_tc_tiling_on_sc=False),
    scratch_types=[
        pltpu.VMEM((_BPW, _T), jnp.int32),
        pltpu.VMEM((_BPW,), jnp.int32),
        pltpu.VMEM((2, _NCHUNK, _L, _L), jnp.float32),
        pltpu.VMEM((_BPW, _L), jnp.float32),
        pltpu.SemaphoreType.DMA,
        pltpu.SemaphoreType.DMA,
    ],
)(_pool_body)


def _head_body(ncls, p_ref, b_ref, o_ref):
    logits = p_ref[...] + b_ref[...]
    col = lax.broadcasted_iota(jnp.int32, logits.shape, 1)
    masked = jnp.where(col < ncls, logits, -1e30)
    m = jnp.max(masked, axis=1, keepdims=True)
    sh = masked - m
    s = jnp.sum(jnp.exp(sh), axis=1, keepdims=True)
    o_ref[...] = sh - jnp.log(s)


def kernel(x, lengths, table, W_out, b_out):
    x = x.astype(jnp.int32)
    lengths = lengths.astype(jnp.int32)
    ncls = W_out.shape[1]

    wp = jnp.zeros((_D, 8, 128), jnp.float32)
    for s in range(8):
        wp = wp.at[:, s, _L * s:_L * s + ncls].set(W_out)
    twp = _proj(jnp.swapaxes(table, 0, 1), wp.reshape(_D, 1024))
    tw = twp.reshape(_TWROWS, _L)

    pooled = _pool(x, lengths, tw)

    bp = jnp.zeros((1, _L), jnp.float32).at[0, :ncls].set(b_out)
    out = pl.pallas_call(
        functools.partial(_head_body, ncls),
        out_shape=jax.ShapeDtypeStruct((_B, _L), jnp.float32),
    )(pooled, bp)
    return out[:, :ncls]


# speedup vs baseline: 5.0100x; 1.1357x over previous
import functools

import jax
import jax.numpy as jnp
from jax import lax
from jax.experimental import pallas as pl
from jax.experimental.pallas import tpu as pltpu
from jax.experimental.pallas import tpu_sc as plsc

_B = 4096
_T = 200
_D = 64
_V = 1000000
_L = 16
_NC = 2
_NS = 16
_NW = _NC * _NS
_BPW = _B // _NW
_NCHUNK = (_T + _L - 1) // _L
_BLK = 16384
_GRID = (_V + _BLK - 1) // _BLK
_PIECE = _BLK // 8
_TWROWS = _GRID * _BLK


def _proj_body(tt_ref, w_ref, o_ref):
    pieces = [
        lax.dot_general(
            tt_ref[:, _PIECE * s:_PIECE * (s + 1)],
            w_ref[:, 128 * s:128 * (s + 1)],
            dimension_numbers=(((0,), (0,)), ((), ())),
            preferred_element_type=jnp.float32)
        for s in range(8)
    ]
    while len(pieces) > 1:
        pieces = [pieces[k] + pieces[k + 1] for k in range(0, len(pieces), 2)]
    o_ref[...] = pieces[0]


_proj = pl.pallas_call(
    _proj_body,
    grid=(_GRID,),
    in_specs=[
        pl.BlockSpec((_D, _BLK), lambda i: (0, i)),
        pl.BlockSpec((_D, 1024), lambda i: (0, 0)),
    ],
    out_specs=pl.BlockSpec((_PIECE, 128), lambda i: (i, 0)),
    out_shape=jax.ShapeDtypeStruct((_GRID * _PIECE, 128), jnp.float32),
    compiler_params=pltpu.CompilerParams(fuse_transposed_lhs_in_matmul=True),
)


def _pool_body(x_hbm, len_hbm, tw_hbm, out_hbm, xv, lenv, dst, outv,
               sem0, sem1):
    wid = lax.axis_index("s") * _NC + lax.axis_index("c")
    base = wid * _BPW
    pltpu.sync_copy(x_hbm.at[pl.ds(base, _BPW)], xv)
    pltpu.sync_copy(len_hbm.at[pl.ds(base, _BPW)], lenv)

    iota16 = lax.broadcasted_iota(jnp.int32, (_L,), 0)
    zerov = jnp.zeros((_L,), jnp.float32)
    sems = (sem0, sem1)

    def issue(b, slot):
        g = b // _L
        j = b - g * _L
        lvec = lenv[pl.ds(g * _L, _L)]
        lb = jnp.max(jnp.where(iota16 == j, lvec, 0))
        nch = (lb + _L - 1) // _L

        def chunk(cb, carry):
            start = jnp.minimum(cb * _L, _T - _L)
            xc = xv[b, pl.ds(start, _L)]
            row16 = (
                ((xc >> 14) << 14)
                | ((xc & 2047) << 3)
                | ((xc & 16383) >> 11))
            pltpu.async_copy(tw_hbm.at[row16], dst.at[slot, cb], sems[slot])
            return carry

        lax.fori_loop(0, nch, chunk, 0)
        return lb

    def finish(b, slot, lb):
        nch = (lb + _L - 1) // _L

        def drain(cb, carry):
            pltpu.make_async_copy(
                tw_hbm.at[pl.ds(0, _L)], dst.at[slot, 0], sems[slot]).wait()
            return carry

        lax.fori_loop(0, nch, drain, 0)

        nfull = jnp.minimum(lb // _L, _NCHUNK - 1)
        rem_off = jnp.where(
            lb >= (_NCHUNK - 1) * _L, (_NCHUNK - 1) * _L - (_T - _L), 0)
        rem_cnt = lb - nfull * _L

        def accum(cb, acc):
            for r in range(_L):
                acc = acc + dst[slot, cb, r, pl.ds(0, _L)]
            return acc

        acc = lax.fori_loop(0, nfull, accum, zerov)

        def accum_tail(r, acc):
            return acc + dst[slot, nfull, rem_off + r, pl.ds(0, _L)]

        acc = lax.fori_loop(0, rem_cnt, accum_tail, acc)

        denv = jnp.maximum(jnp.full((_L,), lb, jnp.int32), 1).astype(
            jnp.float32)
        outv[b, pl.ds(0, _L)] = acc / denv

    lb0 = issue(0, 0)

    def pipe2(i, lb_even):
        b = 2 * i
        lb_odd = issue(b + 1, 1)
        finish(b, 0, lb_even)
        lb_next = issue(b + 2, 0)
        finish(b + 1, 1, lb_odd)
        return lb_next

    lb_last = lax.fori_loop(0, (_BPW - 2) // 2, pipe2, lb0)
    lb_127 = issue(_BPW - 1, 1)
    finish(_BPW - 2, 0, lb_last)
    finish(_BPW - 1, 1, lb_127)

    pltpu.sync_copy(outv, out_hbm.at[pl.ds(base, _BPW)])


_pool = functools.partial(
    pl.kernel,
    out_type=jax.ShapeDtypeStruct((_B, _L), jnp.float32),
    mesh=plsc.VectorSubcoreMesh(core_axis_name="c", subcore_axis_name="s"),
    compiler_params=pltpu.CompilerParams(
        needs_layout_passes=False, use_tc_tiling_on_sc=False),
    scratch_types=[
        pltpu.VMEM((_BPW, _T), jnp.int32),
        pltpu.VMEM((_BPW,), jnp.int32),
        pltpu.VMEM((2, _NCHUNK, _L, _L), jnp.float32),
        pltpu.VMEM((_BPW, _L), jnp.float32),
        pltpu.SemaphoreType.DMA,
        pltpu.SemaphoreType.DMA,
    ],
)(_pool_body)


def _head_body(ncls, p_ref, b_ref, o_ref):
    logits = p_ref[...] + b_ref[...]
    col = lax.broadcasted_iota(jnp.int32, logits.shape, 1)
    masked = jnp.where(col < ncls, logits, -1e30)
    m = jnp.max(masked, axis=1, keepdims=True)
    sh = masked - m
    s = jnp.sum(jnp.exp(sh), axis=1, keepdims=True)
    o_ref[...] = sh - jnp.log(s)


def kernel(x, lengths, table, W_out, b_out):
    x = x.astype(jnp.int32)
    lengths = lengths.astype(jnp.int32)
    ncls = W_out.shape[1]

    wp = jnp.zeros((_D, 8, 128), jnp.float32)
    for s in range(8):
        wp = wp.at[:, s, _L * s:_L * s + ncls].set(W_out)
    twp = _proj(jnp.swapaxes(table, 0, 1), wp.reshape(_D, 1024))
    tw = twp.reshape(_TWROWS, _L)

    pooled = _pool(x, lengths, tw)

    bp = jnp.zeros((1, _L), jnp.float32).at[0, :ncls].set(b_out)
    out = pl.pallas_call(
        functools.partial(_head_body, ncls),
        out_shape=jax.ShapeDtypeStruct((_B, _L), jnp.float32),
    )(pooled, bp)
    return out[:, :ncls]


# speedup vs baseline: 5.2959x; 1.0571x over previous
import functools

import jax
import jax.numpy as jnp
from jax import lax
from jax.experimental import pallas as pl
from jax.experimental.pallas import tpu as pltpu
from jax.experimental.pallas import tpu_sc as plsc

_B = 4096
_T = 200
_D = 64
_V = 1000000
_L = 16
_NC = 2
_NS = 16
_NW = _NC * _NS
_BPW = _B // _NW
_NCHUNK = (_T + _L - 1) // _L
_BLK = 32768
_GRID = (_V + _BLK - 1) // _BLK
_PIECE = _BLK // 8
_TWROWS = _GRID * _BLK


def _proj_body(tt_ref, w_ref, o_ref):
    pieces = [
        lax.dot_general(
            tt_ref[:, _PIECE * s:_PIECE * (s + 1)],
            w_ref[:, 128 * s:128 * (s + 1)],
            dimension_numbers=(((0,), (0,)), ((), ())),
            preferred_element_type=jnp.float32)
        for s in range(8)
    ]
    while len(pieces) > 1:
        pieces = [pieces[k] + pieces[k + 1] for k in range(0, len(pieces), 2)]
    o_ref[...] = pieces[0]


_proj = pl.pallas_call(
    _proj_body,
    grid=(_GRID,),
    in_specs=[
        pl.BlockSpec((_D, _BLK), lambda i: (0, i)),
        pl.BlockSpec((_D, 1024), lambda i: (0, 0)),
    ],
    out_specs=pl.BlockSpec((_PIECE, 128), lambda i: (i, 0)),
    out_shape=jax.ShapeDtypeStruct((_GRID * _PIECE, 128), jnp.float32),
    compiler_params=pltpu.CompilerParams(fuse_transposed_lhs_in_matmul=True),
)


def _pool_body(x_hbm, len_hbm, tw_hbm, out_hbm, xv, lenv, dst, outv,
               sem0, sem1):
    wid = lax.axis_index("s") * _NC + lax.axis_index("c")
    base = wid * _BPW
    pltpu.sync_copy(x_hbm.at[pl.ds(base, _BPW)], xv)
    pltpu.sync_copy(len_hbm.at[pl.ds(base, _BPW)], lenv)

    iota16 = lax.broadcasted_iota(jnp.int32, (_L,), 0)
    zerov = jnp.zeros((_L,), jnp.float32)
    sems = (sem0, sem1)

    def issue(b, slot):
        g = b // _L
        j = b - g * _L
        lvec = lenv[pl.ds(g * _L, _L)]
        lb = jnp.max(jnp.where(iota16 == j, lvec, 0))
        nch = (lb + _L - 1) // _L

        def chunk(cb, carry):
            start = jnp.minimum(cb * _L, _T - _L)
            xc = xv[b, pl.ds(start, _L)]
            row16 = (
                ((xc >> 15) << 15)
                | ((xc & 4095) << 3)
                | ((xc & 32767) >> 12))
            pltpu.async_copy(tw_hbm.at[row16], dst.at[slot, cb], sems[slot])
            return carry

        lax.fori_loop(0, nch, chunk, 0)
        return lb

    def finish(b, slot, lb):
        nch = (lb + _L - 1) // _L

        def drain(cb, carry):
            pltpu.make_async_copy(
                tw_hbm.at[pl.ds(0, _L)], dst.at[slot, 0], sems[slot]).wait()
            return carry

        lax.fori_loop(0, nch, drain, 0)

        nfull = jnp.minimum(lb // _L, _NCHUNK - 1)
        rem_off = jnp.where(
            lb >= (_NCHUNK - 1) * _L, (_NCHUNK - 1) * _L - (_T - _L), 0)
        rem_cnt = lb - nfull * _L

        def accum(cb, acc):
            for r in range(_L):
                acc = acc + dst[slot, cb, r, pl.ds(0, _L)]
            return acc

        acc = lax.fori_loop(0, nfull, accum, zerov)

        def accum_tail(r, acc):
            return acc + dst[slot, nfull, rem_off + r, pl.ds(0, _L)]

        acc = lax.fori_loop(0, rem_cnt, accum_tail, acc)

        denv = jnp.maximum(jnp.full((_L,), lb, jnp.int32), 1).astype(
            jnp.float32)
        outv[b, pl.ds(0, _L)] = acc / denv

    lb0 = issue(0, 0)

    def pipe2(i, lb_even):
        b = 2 * i
        lb_odd = issue(b + 1, 1)
        finish(b, 0, lb_even)
        lb_next = issue(b + 2, 0)
        finish(b + 1, 1, lb_odd)
        return lb_next

    lb_last = lax.fori_loop(0, (_BPW - 2) // 2, pipe2, lb0)
    lb_127 = issue(_BPW - 1, 1)
    finish(_BPW - 2, 0, lb_last)
    finish(_BPW - 1, 1, lb_127)

    pltpu.sync_copy(outv, out_hbm.at[pl.ds(base, _BPW)])


_pool = functools.partial(
    pl.kernel,
    out_type=jax.ShapeDtypeStruct((_B, _L), jnp.float32),
    mesh=plsc.VectorSubcoreMesh(core_axis_name="c", subcore_axis_name="s"),
    compiler_params=pltpu.CompilerParams(
        needs_layout_passes=False, use_tc_tiling_on_sc=False),
    scratch_types=[
        pltpu.VMEM((_BPW, _T), jnp.int32),
        pltpu.VMEM((_BPW,), jnp.int32),
        pltpu.VMEM((2, _NCHUNK, _L, _L), jnp.float32),
        pltpu.VMEM((_BPW, _L), jnp.float32),
        pltpu.SemaphoreType.DMA,
        pltpu.SemaphoreType.DMA,
    ],
)(_pool_body)


def _head_body(ncls, p_ref, b_ref, o_ref):
    logits = p_ref[...] + b_ref[...]
    col = lax.broadcasted_iota(jnp.int32, logits.shape, 1)
    masked = jnp.where(col < ncls, logits, -1e30)
    m = jnp.max(masked, axis=1, keepdims=True)
    sh = masked - m
    s = jnp.sum(jnp.exp(sh), axis=1, keepdims=True)
    o_ref[...] = sh - jnp.log(s)


def kernel(x, lengths, table, W_out, b_out):
    x = x.astype(jnp.int32)
    lengths = lengths.astype(jnp.int32)
    ncls = W_out.shape[1]

    wp = jnp.zeros((_D, 8, 128), jnp.float32)
    for s in range(8):
        wp = wp.at[:, s, _L * s:_L * s + ncls].set(W_out)
    twp = _proj(jnp.swapaxes(table, 0, 1), wp.reshape(_D, 1024))
    tw = twp.reshape(_TWROWS, _L)

    pooled = _pool(x, lengths, tw)

    bp = jnp.zeros((1, _L), jnp.float32).at[0, :ncls].set(b_out)
    out = pl.pallas_call(
        functools.partial(_head_body, ncls),
        out_shape=jax.ShapeDtypeStruct((_B, _L), jnp.float32),
    )(pooled, bp)
    return out[:, :ncls]
